# baseline (speedup 1.0000x reference)
import functools

import jax
import jax.numpy as jnp
from jax import lax
from jax.experimental import pallas as pl
from jax.experimental.pallas import tpu as pltpu
from jax.experimental.pallas import tpu_sc as plsc

N = 10000
E = 320000
D = 128

NC = 2
NS = 16
CHUNK = 128
N_ACC = 10240
ROWS_PER_TILE = N_ACC // NS
E_PAD = 327680
EDGES_PER_TILE = E_PAD // NS
NCHUNK = EDGES_PER_TILE // CHUNK
DUMMY = N

_sc_mesh = plsc.VectorSubcoreMesh(
    core_axis_name="c", subcore_axis_name="s", num_cores=NC, num_subcores=NS
)


@functools.partial(
    pl.kernel,
    out_type=(
        jax.ShapeDtypeStruct((N_ACC, D), jnp.float32),
        jax.ShapeDtypeStruct((N_ACC, D), jnp.float32),
        jax.ShapeDtypeStruct((N_ACC, D), jnp.float32),
    ),
    mesh=_sc_mesh,
    scratch_types=(
        pltpu.VMEM((CHUNK,), jnp.int32),
        pltpu.VMEM((NCHUNK // 2, CHUNK), jnp.int32),
        pltpu.VMEM((CHUNK, D), jnp.float32),
        pltpu.VMEM((CHUNK, D), jnp.float32),
        pltpu.VMEM_SHARED((N_ACC, D), jnp.float32),
        pltpu.SemaphoreType.DMA,
    ),
)
def _sc_deg_perm(dst2_hbm, perm_hbm, p_hbm, ones_hbm, zrows_hbm,
                 dega_out, degb_out, pc_out, idx_v, didx2, rows_v, ones_v,
                 acc_deg, sem):
    c = lax.axis_index("c")
    s = lax.axis_index("s")
    base = s * ROWS_PER_TILE
    half = NCHUNK // 2

    for j in range(ROWS_PER_TILE // CHUNK):
        pltpu.sync_copy(zrows_hbm, acc_deg.at[pl.ds(base + j * CHUNK, CHUNK)])
    pltpu.sync_copy(ones_hbm, ones_v)
    pltpu.sync_copy(dst2_hbm.at[pl.ds((c * NS + s) * half, half)], didx2)
    plsc.subcore_barrier()

    def body(j, _):
        pltpu.sync_copy(ones_v, acc_deg.at[didx2.at[j]], add=True)
        return _

    lax.fori_loop(0, half, body, None)
    plsc.subcore_barrier()

    @pl.when(c == 0)
    def _outa():
        for j in range(ROWS_PER_TILE // CHUNK):
            pltpu.sync_copy(
                acc_deg.at[pl.ds(base + j * CHUNK, CHUNK)],
                dega_out.at[pl.ds(base + j * CHUNK, CHUNK)],
            )

    @pl.when(c == 1)
    def _outb():
        for j in range(ROWS_PER_TILE // CHUNK):
            pltpu.sync_copy(
                acc_deg.at[pl.ds(base + j * CHUNK, CHUNK)],
                degb_out.at[pl.ds(base + j * CHUNK, CHUNK)],
            )

    @pl.when(c == 1)
    def _permgather():
        def body(j, _):
            off = s * ROWS_PER_TILE + j * CHUNK
            pltpu.sync_copy(perm_hbm.at[pl.ds(off, CHUNK)], idx_v)
            pltpu.async_copy(p_hbm.at[idx_v], rows_v, sem).wait()
            pltpu.sync_copy(rows_v, pc_out.at[pl.ds(off, CHUNK)])
            return _

        lax.fori_loop(0, ROWS_PER_TILE // CHUNK, body, None)


@functools.partial(
    pl.kernel,
    out_type=(
        jax.ShapeDtypeStruct((N_ACC, D), jnp.float32),
        jax.ShapeDtypeStruct((N_ACC, D), jnp.float32),
    ),
    mesh=_sc_mesh,
    scratch_types=(
        pltpu.VMEM((CHUNK,), jnp.int32),
        pltpu.VMEM((CHUNK,), jnp.int32),
        pltpu.VMEM((CHUNK,), jnp.int32),
        pltpu.VMEM((CHUNK,), jnp.int32),
        pltpu.VMEM((CHUNK, D), jnp.float32),
        pltpu.VMEM((CHUNK, D), jnp.float32),
        pltpu.VMEM_SHARED((N_ACC, D), jnp.float32),
        pltpu.SemaphoreType.DMA,
        pltpu.SemaphoreType.DMA,
        pltpu.SemaphoreType.DMA,
        pltpu.SemaphoreType.DMA,
    ),
)
def _sc_edge_pass(src1_hbm, dst1_hbm, za_hbm, zb_hbm, zrows_hbm,
                  sa_out, sb_out, sidx0, sidx1, didx0, didx1, rows0, rows1,
                  acc, semg0, semg1, sems0, sems1):
    c = lax.axis_index("c")
    s = lax.axis_index("s")

    def do_pass(table, out):
        base = s * ROWS_PER_TILE
        cbase = s * NCHUNK
        for j in range(ROWS_PER_TILE // CHUNK):
            pltpu.sync_copy(zrows_hbm, acc.at[pl.ds(base + j * CHUNK, CHUNK)])
        plsc.subcore_barrier()

        def load_idx(j, sidx, didx):
            off = (cbase + j) * CHUNK
            pltpu.sync_copy(src1_hbm.at[pl.ds(off, CHUNK)], sidx)
            pltpu.sync_copy(dst1_hbm.at[pl.ds(off, CHUNK)], didx)

        load_idx(0, sidx0, didx0)
        pltpu.async_copy(table.at[sidx0], rows0, semg0)
        load_idx(1, sidx1, didx1)
        pltpu.async_copy(table.at[sidx1], rows1, semg1)

        def body(i, _):
            j0 = 2 * i
            j1 = j0 + 1
            pltpu.make_async_copy(table.at[sidx0], rows0, semg0).wait()
            pltpu.async_copy(rows0, acc.at[didx0], sems0, add=True)
            pltpu.make_async_copy(table.at[sidx1], rows1, semg1).wait()
            pltpu.async_copy(rows1, acc.at[didx1], sems1, add=True)

            @pl.when(j0 + 2 < NCHUNK)
            def _():
                pltpu.make_async_copy(rows0, acc.at[didx0], sems0).wait()
                load_idx(j0 + 2, sidx0, didx0)
                pltpu.async_copy(table.at[sidx0], rows0, semg0)

            @pl.when(j1 + 2 < NCHUNK)
            def _():
                pltpu.make_async_copy(rows1, acc.at[didx1], sems1).wait()
                load_idx(j1 + 2, sidx1, didx1)
                pltpu.async_copy(table.at[sidx1], rows1, semg1)

            return _

        lax.fori_loop(0, NCHUNK // 2, body, None)
        pltpu.make_async_copy(rows0, acc.at[didx0], sems0).wait()
        pltpu.make_async_copy(rows1, acc.at[didx1], sems1).wait()
        plsc.subcore_barrier()
        for j in range(ROWS_PER_TILE // CHUNK):
            pltpu.sync_copy(
                acc.at[pl.ds(base + j * CHUNK, CHUNK)],
                out.at[pl.ds(base + j * CHUNK, CHUNK)],
            )

    @pl.when(c == 0)
    def _a():
        do_pass(za_hbm, sa_out)

    @pl.when(c == 1)
    def _b():
        do_pass(zb_hbm, sb_out)


BM = 1000
GRID = N // BM


def _tc_matmul_body(x_ref, w_ref, o_ref):
    o_ref[...] = jnp.dot(x_ref[...], w_ref[...], preferred_element_type=jnp.float32)


def _tc_matmul(x, w):
    return pl.pallas_call(
        _tc_matmul_body,
        grid=(GRID,),
        in_specs=[
            pl.BlockSpec((BM, D), lambda i: (i, 0)),
            pl.BlockSpec((D, D), lambda i: (0, 0)),
        ],
        out_specs=pl.BlockSpec((BM, D), lambda i: (i, 0)),
        out_shape=jax.ShapeDtypeStruct((N, D), jnp.float32),
    )(x, w)


def _dinv_of(dega_blk, degb_blk):
    return lax.rsqrt(dega_blk[:, 0:1] + degb_blk[:, 0:1] + 1.0)


def _tc_scale_body(p_ref, pc_ref, dega_ref, degb_ref, za_ref, zb_ref):
    dinv = _dinv_of(dega_ref[...], degb_ref[...])
    za_ref[...] = p_ref[...] * dinv
    zb_ref[...] = pc_ref[...] * dinv


def _tc_scale(p, pc, dega, degb):
    return pl.pallas_call(
        _tc_scale_body,
        grid=(GRID,),
        in_specs=[
            pl.BlockSpec((BM, D), lambda i: (i, 0)),
            pl.BlockSpec((BM, D), lambda i: (i, 0)),
            pl.BlockSpec((BM, D), lambda i: (i, 0)),
            pl.BlockSpec((BM, D), lambda i: (i, 0)),
        ],
        out_specs=[
            pl.BlockSpec((BM, D), lambda i: (i, 0)),
            pl.BlockSpec((BM, D), lambda i: (i, 0)),
        ],
        out_shape=[
            jax.ShapeDtypeStruct((N, D), jnp.float32),
            jax.ShapeDtypeStruct((N, D), jnp.float32),
        ],
    )(p, pc, dega, degb)


def _tc_layer_body(sa_ref, sb_ref, za_ref, zb_ref, dega_ref, degb_ref, b1_ref,
                   w2_ref, oa_ref, ob_ref):
    dinv = _dinv_of(dega_ref[...], degb_ref[...])
    h1a = jax.nn.relu(dinv * (sa_ref[...] + za_ref[...]) + b1_ref[...])
    h1b = jax.nn.relu(dinv * (sb_ref[...] + zb_ref[...]) + b1_ref[...])
    oa_ref[...] = dinv * jnp.dot(h1a, w2_ref[...], preferred_element_type=jnp.float32)
    ob_ref[...] = dinv * jnp.dot(h1b, w2_ref[...], preferred_element_type=jnp.float32)


def _tc_layer(sa, sb, za, zb, dega, degb, b1, w2):
    return pl.pallas_call(
        _tc_layer_body,
        grid=(GRID,),
        in_specs=[
            pl.BlockSpec((BM, D), lambda i: (i, 0)),
            pl.BlockSpec((BM, D), lambda i: (i, 0)),
            pl.BlockSpec((BM, D), lambda i: (i, 0)),
            pl.BlockSpec((BM, D), lambda i: (i, 0)),
            pl.BlockSpec((BM, D), lambda i: (i, 0)),
            pl.BlockSpec((BM, D), lambda i: (i, 0)),
            pl.BlockSpec((1, D), lambda i: (0, 0)),
            pl.BlockSpec((D, D), lambda i: (0, 0)),
        ],
        out_specs=[
            pl.BlockSpec((BM, D), lambda i: (i, 0)),
            pl.BlockSpec((BM, D), lambda i: (i, 0)),
        ],
        out_shape=[
            jax.ShapeDtypeStruct((N, D), jnp.float32),
            jax.ShapeDtypeStruct((N, D), jnp.float32),
        ],
    )(sa, sb, za, zb, dega, degb, b1, w2)


def _tc_final_h_body(sa_ref, sb_ref, za_ref, zb_ref, dega_ref, degb_ref,
                     b2_ref, ha_ref, hb_ref, cs_ref):
    i = pl.program_id(0)
    dinv = _dinv_of(dega_ref[...], degb_ref[...])
    ha = dinv * (sa_ref[...] + za_ref[...]) + b2_ref[...]
    hb = dinv * (sb_ref[...] + zb_ref[...]) + b2_ref[...]
    ha_ref[...] = ha
    hb_ref[...] = hb

    @pl.when(i == 0)
    def _():
        cs_ref[...] = jnp.zeros_like(cs_ref)

    cs_ref[...] += jnp.sum(ha, axis=0, keepdims=True)


def _tc_final_h(sa, sb, za, zb, dega, degb, b2):
    return pl.pallas_call(
        _tc_final_h_body,
        grid=(GRID,),
        in_specs=[
            pl.BlockSpec((BM, D), lambda i: (i, 0)),
            pl.BlockSpec((BM, D), lambda i: (i, 0)),
            pl.BlockSpec((BM, D), lambda i: (i, 0)),
            pl.BlockSpec((BM, D), lambda i: (i, 0)),
            pl.BlockSpec((BM, D), lambda i: (i, 0)),
            pl.BlockSpec((BM, D), lambda i: (i, 0)),
            pl.BlockSpec((1, D), lambda i: (0, 0)),
        ],
        out_specs=[
            pl.BlockSpec((BM, D), lambda i: (i, 0)),
            pl.BlockSpec((BM, D), lambda i: (i, 0)),
            pl.BlockSpec((1, D), lambda i: (0, 0)),
        ],
        out_shape=[
            jax.ShapeDtypeStruct((N, D), jnp.float32),
            jax.ShapeDtypeStruct((N, D), jnp.float32),
            jax.ShapeDtypeStruct((1, D), jnp.float32),
        ],
    )(sa, sb, za, zb, dega, degb, b2)


def _tc_score_body(ha_ref, hb_ref, wb_ref, cs_ref, bb_ref, pos_ref, neg_ref):
    s_row = jax.nn.sigmoid(cs_ref[...] / float(N))
    ta = jnp.dot(ha_ref[...], wb_ref[...], preferred_element_type=jnp.float32)
    tb = jnp.dot(hb_ref[...], wb_ref[...], preferred_element_type=jnp.float32)
    pos_ref[...] = jnp.sum(ta * s_row, axis=1, keepdims=True) + bb_ref[...]
    neg_ref[...] = jnp.sum(tb * s_row, axis=1, keepdims=True) + bb_ref[...]


def _tc_score(ha, hb, wb0, cs, bb):
    return pl.pallas_call(
        _tc_score_body,
        grid=(GRID,),
        in_specs=[
            pl.BlockSpec((BM, D), lambda i: (i, 0)),
            pl.BlockSpec((BM, D), lambda i: (i, 0)),
            pl.BlockSpec((D, D), lambda i: (0, 0)),
            pl.BlockSpec((1, D), lambda i: (0, 0)),
            pl.BlockSpec((1, 1), lambda i: (0, 0)),
        ],
        out_specs=[
            pl.BlockSpec((BM, 1), lambda i: (i, 0)),
            pl.BlockSpec((BM, 1), lambda i: (i, 0)),
        ],
        out_shape=[
            jax.ShapeDtypeStruct((N, 1), jnp.float32),
            jax.ShapeDtypeStruct((N, 1), jnp.float32),
        ],
    )(ha, hb, wb0, cs, bb)


def kernel(x, edge_index, W1, b1, W2, b2, Wb, bb, perm):
    src = edge_index[0].astype(jnp.int32)
    dst = edge_index[1].astype(jnp.int32)
    pad_e = E_PAD - E
    src1 = jnp.concatenate([src, jnp.zeros((pad_e,), jnp.int32)])
    dst1 = jnp.concatenate([dst, jnp.full((pad_e,), DUMMY, jnp.int32)])
    dst2 = dst1.reshape(E_PAD // CHUNK, CHUNK)
    perm_pad = jnp.concatenate(
        [perm.astype(jnp.int32), jnp.zeros((N_ACC - N,), jnp.int32)]
    )
    ones = jnp.ones((CHUNK, D), jnp.float32)
    zrows = jnp.zeros((CHUNK, D), jnp.float32)
    b1r = b1.reshape(1, D)
    b2r = b2.reshape(1, D)
    wb0 = Wb.reshape(D, D)
    bbr = bb.reshape(1, 1)

    p = _tc_matmul(x, W1)
    dega, degb, pc = _sc_deg_perm(dst2, perm_pad, p, ones, zrows)
    pc = pc[:N]
    dega = dega[:N]
    degb = degb[:N]
    za, zb = _tc_scale(p, pc, dega, degb)
    sa, sb = _sc_edge_pass(src1, dst1, za, zb, zrows)
    z2a, z2b = _tc_layer(sa[:N], sb[:N], za, zb, dega, degb, b1r, W2)
    s2a, s2b = _sc_edge_pass(src1, dst1, z2a, z2b, zrows)
    ha, hb, cs = _tc_final_h(s2a[:N], s2b[:N], z2a, z2b, dega, degb, b2r)
    pos, neg = _tc_score(ha, hb, wb0, cs, bbr)
    return (pos, neg)


# speedup vs baseline: 1.1526x; 1.1526x over previous
import functools

import jax
import jax.numpy as jnp
from jax import lax
from jax.experimental import pallas as pl
from jax.experimental.pallas import tpu as pltpu
from jax.experimental.pallas import tpu_sc as plsc

N = 10000
E = 320000
D = 128

NC = 2
NS = 16
CHUNK = 128
N_ACC = 10240
ROWS_PER_TILE = N_ACC // NS
E_PAD = 327680
EDGES_PER_TILE = E_PAD // NS
NCHUNK = EDGES_PER_TILE // CHUNK
DUMMY = N
IBLK = 16

_sc_mesh = plsc.VectorSubcoreMesh(
    core_axis_name="c", subcore_axis_name="s", num_cores=NC, num_subcores=NS
)


@functools.partial(
    pl.kernel,
    out_type=(
        jax.ShapeDtypeStruct((N_ACC, D), jnp.float32),
        jax.ShapeDtypeStruct((N_ACC, D), jnp.float32),
        jax.ShapeDtypeStruct((N_ACC, D), jnp.float32),
    ),
    mesh=_sc_mesh,
    scratch_types=(
        pltpu.VMEM((CHUNK,), jnp.int32),
        pltpu.VMEM((NCHUNK // 2, CHUNK), jnp.int32),
        pltpu.VMEM((CHUNK, D), jnp.float32),
        pltpu.VMEM((CHUNK, D), jnp.float32),
        pltpu.VMEM_SHARED((N_ACC, D), jnp.float32),
        pltpu.SemaphoreType.DMA,
    ),
)
def _sc_deg_perm(dst2_hbm, perm_hbm, p_hbm, ones_hbm, zrows_hbm,
                 dega_out, degb_out, pc_out, idx_v, didx2, rows_v, ones_v,
                 acc_deg, sem):
    c = lax.axis_index("c")
    s = lax.axis_index("s")
    base = s * ROWS_PER_TILE
    half = NCHUNK // 2

    for j in range(ROWS_PER_TILE // CHUNK):
        pltpu.sync_copy(zrows_hbm, acc_deg.at[pl.ds(base + j * CHUNK, CHUNK)])
    pltpu.sync_copy(ones_hbm, ones_v)
    pltpu.sync_copy(dst2_hbm.at[pl.ds((c * NS + s) * half, half)], didx2)
    plsc.subcore_barrier()

    def body(j, _):
        pltpu.sync_copy(ones_v, acc_deg.at[didx2.at[j]], add=True)
        return _

    lax.fori_loop(0, half, body, None)
    plsc.subcore_barrier()

    @pl.when(c == 0)
    def _outa():
        for j in range(ROWS_PER_TILE // CHUNK):
            pltpu.sync_copy(
                acc_deg.at[pl.ds(base + j * CHUNK, CHUNK)],
                dega_out.at[pl.ds(base + j * CHUNK, CHUNK)],
            )

    @pl.when(c == 1)
    def _outb():
        for j in range(ROWS_PER_TILE // CHUNK):
            pltpu.sync_copy(
                acc_deg.at[pl.ds(base + j * CHUNK, CHUNK)],
                degb_out.at[pl.ds(base + j * CHUNK, CHUNK)],
            )

    @pl.when(c == 1)
    def _permgather():
        def body(j, _):
            off = s * ROWS_PER_TILE + j * CHUNK
            pltpu.sync_copy(perm_hbm.at[pl.ds(off, CHUNK)], idx_v)
            pltpu.async_copy(p_hbm.at[idx_v], rows_v, sem).wait()
            pltpu.sync_copy(rows_v, pc_out.at[pl.ds(off, CHUNK)])
            return _

        lax.fori_loop(0, ROWS_PER_TILE // CHUNK, body, None)


@functools.partial(
    pl.kernel,
    out_type=(
        jax.ShapeDtypeStruct((N_ACC, D), jnp.float32),
        jax.ShapeDtypeStruct((N_ACC, D), jnp.float32),
    ),
    mesh=_sc_mesh,
    scratch_types=(
        pltpu.VMEM((IBLK, CHUNK), jnp.int32),
        pltpu.VMEM((IBLK, CHUNK), jnp.int32),
        pltpu.VMEM((CHUNK, D), jnp.float32),
        pltpu.VMEM((CHUNK, D), jnp.float32),
        pltpu.VMEM_SHARED((N_ACC, D), jnp.float32),
        pltpu.SemaphoreType.DMA,
        pltpu.SemaphoreType.DMA,
    ),
)
def _sc_edge_pass(src2_hbm, dst2_hbm, za_hbm, zb_hbm, zrows_hbm,
                  sa_out, sb_out, sidx2, didx2, rows0, rows1,
                  acc, sem0, sem1):
    c = lax.axis_index("c")
    s = lax.axis_index("s")

    def do_pass(table, out):
        base = s * ROWS_PER_TILE
        cbase = s * NCHUNK
        for j in range(ROWS_PER_TILE // CHUNK):
            pltpu.sync_copy(zrows_hbm, acc.at[pl.ds(base + j * CHUNK, CHUNK)])
        plsc.subcore_barrier()

        def block(k, _):
            pltpu.sync_copy(src2_hbm.at[pl.ds(cbase + k * IBLK, IBLK)], sidx2)
            pltpu.sync_copy(dst2_hbm.at[pl.ds(cbase + k * IBLK, IBLK)], didx2)
            pltpu.async_copy(table.at[sidx2.at[0]], rows0, sem0)

            def body(m, _):
                j0 = 2 * m
                j1 = j0 + 1
                pltpu.async_copy(table.at[sidx2.at[j1]], rows1, sem1)
                pltpu.make_async_copy(table.at[sidx2.at[j0]], rows0, sem0).wait()
                pltpu.sync_copy(rows0, acc.at[didx2.at[j0]], add=True)

                @pl.when(j0 + 2 < IBLK)
                def _():
                    pltpu.async_copy(table.at[sidx2.at[j0 + 2]], rows0, sem0)

                pltpu.make_async_copy(table.at[sidx2.at[j1]], rows1, sem1).wait()
                pltpu.sync_copy(rows1, acc.at[didx2.at[j1]], add=True)
                return _

            lax.fori_loop(0, IBLK // 2, body, None)
            return _

        lax.fori_loop(0, NCHUNK // IBLK, block, None)
        plsc.subcore_barrier()
        for j in range(ROWS_PER_TILE // CHUNK):
            pltpu.sync_copy(
                acc.at[pl.ds(base + j * CHUNK, CHUNK)],
                out.at[pl.ds(base + j * CHUNK, CHUNK)],
            )

    @pl.when(c == 0)
    def _a():
        do_pass(za_hbm, sa_out)

    @pl.when(c == 1)
    def _b():
        do_pass(zb_hbm, sb_out)


BM = 1000
GRID = N // BM


def _tc_matmul_body(x_ref, w_ref, o_ref):
    o_ref[...] = jnp.dot(x_ref[...], w_ref[...], preferred_element_type=jnp.float32)


def _tc_matmul(x, w):
    return pl.pallas_call(
        _tc_matmul_body,
        grid=(GRID,),
        in_specs=[
            pl.BlockSpec((BM, D), lambda i: (i, 0)),
            pl.BlockSpec((D, D), lambda i: (0, 0)),
        ],
        out_specs=pl.BlockSpec((BM, D), lambda i: (i, 0)),
        out_shape=jax.ShapeDtypeStruct((N, D), jnp.float32),
    )(x, w)


def _dinv_of(dega_blk, degb_blk):
    return lax.rsqrt(dega_blk[:, 0:1] + degb_blk[:, 0:1] + 1.0)


def _tc_scale_body(p_ref, pc_ref, dega_ref, degb_ref, za_ref, zb_ref):
    dinv = _dinv_of(dega_ref[...], degb_ref[...])
    za_ref[...] = p_ref[...] * dinv
    zb_ref[...] = pc_ref[...] * dinv


def _tc_scale(p, pc, dega, degb):
    return pl.pallas_call(
        _tc_scale_body,
        grid=(GRID,),
        in_specs=[
            pl.BlockSpec((BM, D), lambda i: (i, 0)),
            pl.BlockSpec((BM, D), lambda i: (i, 0)),
            pl.BlockSpec((BM, D), lambda i: (i, 0)),
            pl.BlockSpec((BM, D), lambda i: (i, 0)),
        ],
        out_specs=[
            pl.BlockSpec((BM, D), lambda i: (i, 0)),
            pl.BlockSpec((BM, D), lambda i: (i, 0)),
        ],
        out_shape=[
            jax.ShapeDtypeStruct((N, D), jnp.float32),
            jax.ShapeDtypeStruct((N, D), jnp.float32),
        ],
    )(p, pc, dega, degb)


def _tc_layer_body(sa_ref, sb_ref, za_ref, zb_ref, dega_ref, degb_ref, b1_ref,
                   w2_ref, oa_ref, ob_ref):
    dinv = _dinv_of(dega_ref[...], degb_ref[...])
    h1a = jax.nn.relu(dinv * (sa_ref[...] + za_ref[...]) + b1_ref[...])
    h1b = jax.nn.relu(dinv * (sb_ref[...] + zb_ref[...]) + b1_ref[...])
    oa_ref[...] = dinv * jnp.dot(h1a, w2_ref[...], preferred_element_type=jnp.float32)
    ob_ref[...] = dinv * jnp.dot(h1b, w2_ref[...], preferred_element_type=jnp.float32)


def _tc_layer(sa, sb, za, zb, dega, degb, b1, w2):
    return pl.pallas_call(
        _tc_layer_body,
        grid=(GRID,),
        in_specs=[
            pl.BlockSpec((BM, D), lambda i: (i, 0)),
            pl.BlockSpec((BM, D), lambda i: (i, 0)),
            pl.BlockSpec((BM, D), lambda i: (i, 0)),
            pl.BlockSpec((BM, D), lambda i: (i, 0)),
            pl.BlockSpec((BM, D), lambda i: (i, 0)),
            pl.BlockSpec((BM, D), lambda i: (i, 0)),
            pl.BlockSpec((1, D), lambda i: (0, 0)),
            pl.BlockSpec((D, D), lambda i: (0, 0)),
        ],
        out_specs=[
            pl.BlockSpec((BM, D), lambda i: (i, 0)),
            pl.BlockSpec((BM, D), lambda i: (i, 0)),
        ],
        out_shape=[
            jax.ShapeDtypeStruct((N, D), jnp.float32),
            jax.ShapeDtypeStruct((N, D), jnp.float32),
        ],
    )(sa, sb, za, zb, dega, degb, b1, w2)


def _tc_final_h_body(sa_ref, sb_ref, za_ref, zb_ref, dega_ref, degb_ref,
                     b2_ref, ha_ref, hb_ref, cs_ref):
    i = pl.program_id(0)
    dinv = _dinv_of(dega_ref[...], degb_ref[...])
    ha = dinv * (sa_ref[...] + za_ref[...]) + b2_ref[...]
    hb = dinv * (sb_ref[...] + zb_ref[...]) + b2_ref[...]
    ha_ref[...] = ha
    hb_ref[...] = hb

    @pl.when(i == 0)
    def _():
        cs_ref[...] = jnp.zeros_like(cs_ref)

    cs_ref[...] += jnp.sum(ha, axis=0, keepdims=True)


def _tc_final_h(sa, sb, za, zb, dega, degb, b2):
    return pl.pallas_call(
        _tc_final_h_body,
        grid=(GRID,),
        in_specs=[
            pl.BlockSpec((BM, D), lambda i: (i, 0)),
            pl.BlockSpec((BM, D), lambda i: (i, 0)),
            pl.BlockSpec((BM, D), lambda i: (i, 0)),
            pl.BlockSpec((BM, D), lambda i: (i, 0)),
            pl.BlockSpec((BM, D), lambda i: (i, 0)),
            pl.BlockSpec((BM, D), lambda i: (i, 0)),
            pl.BlockSpec((1, D), lambda i: (0, 0)),
        ],
        out_specs=[
            pl.BlockSpec((BM, D), lambda i: (i, 0)),
            pl.BlockSpec((BM, D), lambda i: (i, 0)),
            pl.BlockSpec((1, D), lambda i: (0, 0)),
        ],
        out_shape=[
            jax.ShapeDtypeStruct((N, D), jnp.float32),
            jax.ShapeDtypeStruct((N, D), jnp.float32),
            jax.ShapeDtypeStruct((1, D), jnp.float32),
        ],
    )(sa, sb, za, zb, dega, degb, b2)


def _tc_score_body(ha_ref, hb_ref, wb_ref, cs_ref, bb_ref, pos_ref, neg_ref):
    s_row = jax.nn.sigmoid(cs_ref[...] / float(N))
    ta = jnp.dot(ha_ref[...], wb_ref[...], preferred_element_type=jnp.float32)
    tb = jnp.dot(hb_ref[...], wb_ref[...], preferred_element_type=jnp.float32)
    pos_ref[...] = jnp.sum(ta * s_row, axis=1, keepdims=True) + bb_ref[...]
    neg_ref[...] = jnp.sum(tb * s_row, axis=1, keepdims=True) + bb_ref[...]


def _tc_score(ha, hb, wb0, cs, bb):
    return pl.pallas_call(
        _tc_score_body,
        grid=(GRID,),
        in_specs=[
            pl.BlockSpec((BM, D), lambda i: (i, 0)),
            pl.BlockSpec((BM, D), lambda i: (i, 0)),
            pl.BlockSpec((D, D), lambda i: (0, 0)),
            pl.BlockSpec((1, D), lambda i: (0, 0)),
            pl.BlockSpec((1, 1), lambda i: (0, 0)),
        ],
        out_specs=[
            pl.BlockSpec((BM, 1), lambda i: (i, 0)),
            pl.BlockSpec((BM, 1), lambda i: (i, 0)),
        ],
        out_shape=[
            jax.ShapeDtypeStruct((N, 1), jnp.float32),
            jax.ShapeDtypeStruct((N, 1), jnp.float32),
        ],
    )(ha, hb, wb0, cs, bb)


def kernel(x, edge_index, W1, b1, W2, b2, Wb, bb, perm):
    src = edge_index[0].astype(jnp.int32)
    dst = edge_index[1].astype(jnp.int32)
    pad_e = E_PAD - E
    src2 = jnp.concatenate([src, jnp.zeros((pad_e,), jnp.int32)]).reshape(
        E_PAD // CHUNK, CHUNK)
    dst2 = jnp.concatenate([dst, jnp.full((pad_e,), DUMMY, jnp.int32)]).reshape(
        E_PAD // CHUNK, CHUNK)
    perm_pad = jnp.concatenate(
        [perm.astype(jnp.int32), jnp.zeros((N_ACC - N,), jnp.int32)]
    )
    ones = jnp.ones((CHUNK, D), jnp.float32)
    zrows = jnp.zeros((CHUNK, D), jnp.float32)
    b1r = b1.reshape(1, D)
    b2r = b2.reshape(1, D)
    wb0 = Wb.reshape(D, D)
    bbr = bb.reshape(1, 1)

    p = _tc_matmul(x, W1)
    dega, degb, pc = _sc_deg_perm(dst2, perm_pad, p, ones, zrows)
    pc = pc[:N]
    dega = dega[:N]
    degb = degb[:N]
    za, zb = _tc_scale(p, pc, dega, degb)
    sa, sb = _sc_edge_pass(src2, dst2, za, zb, zrows)
    z2a, z2b = _tc_layer(sa[:N], sb[:N], za, zb, dega, degb, b1r, W2)
    s2a, s2b = _sc_edge_pass(src2, dst2, z2a, z2b, zrows)
    ha, hb, cs = _tc_final_h(s2a[:N], s2b[:N], z2a, z2b, dega, degb, b2r)
    pos, neg = _tc_score(ha, hb, wb0, cs, bbr)
    return (pos, neg)
